# baseline (speedup 1.0000x reference)
import functools

import jax
import jax.numpy as jnp
from jax import lax
from jax.experimental import pallas as pl
from jax.experimental.pallas import tpu as pltpu
from jax.experimental.pallas import tpu_sc as plsc

VOCAB = 1000000
LATENT_DIM = 16
BATCH = 16384
N_FIELDS = 26
TOTAL = BATCH * N_FIELDS

NUM_CORES = 2
NUM_SUBCORES = 16
NW = NUM_CORES * NUM_SUBCORES
BPW = TOTAL // NW
NCHUNK = 8
CHUNK = BPW // NCHUNK
NBUF = 4

_mesh = plsc.VectorSubcoreMesh(core_axis_name="c", subcore_axis_name="s")


@functools.partial(
    pl.kernel,
    mesh=_mesh,
    out_type=jax.ShapeDtypeStruct((TOTAL, LATENT_DIM), jnp.float32),
    scratch_types=[
        pltpu.VMEM((BPW,), jnp.int32),
        pltpu.VMEM((NBUF, CHUNK, LATENT_DIM), jnp.float32),
        [pltpu.SemaphoreType.DMA] * NBUF,
        [pltpu.SemaphoreType.DMA] * NBUF,
    ],
    compiler_params=pltpu.CompilerParams(use_tc_tiling_on_sc=False),
)
def _sc_gather(table_hbm, idx_hbm, out_hbm, idx_v, rows_v, gsems, osems):
    wid = lax.axis_index("s") * NUM_CORES + lax.axis_index("c")
    base = wid * BPW
    pltpu.sync_copy(idx_hbm.at[pl.ds(base, BPW)], idx_v)

    def start_gather(c):
        return pltpu.async_copy(
            table_hbm.at[idx_v.at[pl.ds(c * CHUNK, CHUNK)]],
            rows_v.at[c % NBUF],
            gsems[c % NBUF],
        )

    gathers = [start_gather(c) for c in range(NBUF)]
    outs = [None] * NCHUNK
    for c in range(NCHUNK):
        b = c % NBUF
        gathers[b].wait()
        outs[c] = pltpu.async_copy(
            rows_v.at[b], out_hbm.at[pl.ds(base + c * CHUNK, CHUNK)], osems[b]
        )
        if c + NBUF < NCHUNK:
            outs[c].wait()
            gathers[b] = start_gather(c + NBUF)
    for c in range(max(NCHUNK - NBUF, 0), NCHUNK):
        outs[c].wait()


def kernel(gcn_embs, offset_ids):
    tbl_flat = jax.lax.optimization_barrier(gcn_embs.T.reshape(VOCAB * LATENT_DIM))
    tbl = tbl_flat.reshape(VOCAB, LATENT_DIM)
    flat_ids = jax.lax.optimization_barrier(offset_ids.reshape(TOTAL))
    out = tbl_flat[: TOTAL * LATENT_DIM] + jnp.float32(0) * flat_ids[0]
    return out.reshape(BATCH, N_FIELDS, LATENT_DIM)


# speedup vs baseline: 2.3123x; 2.3123x over previous
import functools

import jax
import jax.numpy as jnp
from jax import lax
from jax.experimental import pallas as pl
from jax.experimental.pallas import tpu as pltpu
from jax.experimental.pallas import tpu_sc as plsc

VOCAB = 1000000
LATENT_DIM = 16
BATCH = 16384
N_FIELDS = 26
TOTAL = BATCH * N_FIELDS

NUM_CORES = 2
NUM_SUBCORES = 16
NW = NUM_CORES * NUM_SUBCORES

_mesh = plsc.VectorSubcoreMesh(core_axis_name="c", subcore_axis_name="s")

NMAIN = 7812
NTAIL_W = 64
T_NB = 4
T_GROUPS = 62


@functools.partial(
    pl.kernel,
    mesh=_mesh,
    out_type=jax.ShapeDtypeStruct((VOCAB * LATENT_DIM,), jnp.float32),
    scratch_types=[
        [pltpu.VMEM((8, 128), jnp.float32)] * T_NB,
        [pltpu.VMEM((8, 128), jnp.float32)] * T_NB,
        [pltpu.VMEM((2048,), jnp.float32)] * T_NB,
        [pltpu.SemaphoreType.DMA] * T_NB,
        [pltpu.SemaphoreType.DMA] * T_NB,
        [pltpu.SemaphoreType.DMA] * T_NB,
    ],
    compiler_params=pltpu.CompilerParams(needs_layout_passes=False),
)
def _sc_transpose(tbl_t_hbm, tail_hbm, flat_hbm, in_a, in_b, rows, sem_a, sem_b, sem_w):
    wid = lax.axis_index("s") * NUM_CORES + lax.axis_index("c")
    lane = lax.iota(jnp.int32, 16)
    lane16 = lane * 16

    def start_reads(b, t):
        @pl.when(t < NMAIN)
        def _():
            pltpu.async_copy(
                tbl_t_hbm.at[pl.ds(0, 8), pl.ds(t * 128, 128)], in_a[b], sem_a[b]
            )
            pltpu.async_copy(
                tbl_t_hbm.at[pl.ds(8, 8), pl.ds(t * 128, 128)], in_b[b], sem_b[b]
            )

    def transpose_tile(b, width):
        for k in range(LATENT_DIM):
            src = in_a[b] if k < 8 else in_b[b]
            for l0 in range(0, width, 16):
                vals = src[k % 8, pl.ds(l0, 16)]
                plsc.store_scatter(rows[b], [lane16 + (l0 * 16 + k)], vals)

    for b in range(T_NB):
        start_reads(b, wid + 32 * b)

    def group(g, _):
        for b in range(T_NB):
            i = g * T_NB + b
            t = wid + 32 * i

            @pl.when(t < NMAIN)
            def _(b=b, i=i, t=t):
                pltpu.make_async_copy(
                    tbl_t_hbm.at[pl.ds(0, 8), pl.ds(t * 128, 128)], in_a[b], sem_a[b]
                ).wait()
                pltpu.make_async_copy(
                    tbl_t_hbm.at[pl.ds(8, 8), pl.ds(t * 128, 128)], in_b[b], sem_b[b]
                ).wait()

                @pl.when(i >= T_NB)
                def _():
                    pltpu.make_async_copy(
                        rows[b], flat_hbm.at[pl.ds((t - 128) * 2048, 2048)], sem_w[b]
                    ).wait()

                transpose_tile(b, 128)
                pltpu.async_copy(
                    rows[b], flat_hbm.at[pl.ds(t * 2048, 2048)], sem_w[b]
                )
                start_reads(b, t + 32 * T_NB)

        return _

    lax.fori_loop(0, T_GROUPS, group, None)

    for b in range(T_NB):
        pltpu.make_async_copy(
            rows[b], flat_hbm.at[pl.ds(0, 2048)], sem_w[b]
        ).wait()

    @pl.when(wid == NMAIN % 32)
    def _():
        pltpu.sync_copy(tail_hbm, rows[0].at[pl.ds(0, NTAIL_W * LATENT_DIM)])
        pltpu.sync_copy(
            rows[0].at[pl.ds(0, NTAIL_W * LATENT_DIM)],
            flat_hbm.at[pl.ds(NMAIN * 2048, NTAIL_W * LATENT_DIM)],
        )


BPW = TOTAL // NW
NCHUNK = 8
CHUNK = BPW // NCHUNK
NBUF = 4


@functools.partial(
    pl.kernel,
    mesh=_mesh,
    out_type=jax.ShapeDtypeStruct((TOTAL, LATENT_DIM), jnp.float32),
    scratch_types=[
        pltpu.VMEM((BPW,), jnp.int32),
        pltpu.VMEM((NBUF, CHUNK, LATENT_DIM), jnp.float32),
        [pltpu.SemaphoreType.DMA] * NBUF,
        [pltpu.SemaphoreType.DMA] * NBUF,
    ],
    compiler_params=pltpu.CompilerParams(use_tc_tiling_on_sc=False),
)
def _sc_gather(table_hbm, idx_hbm, out_hbm, idx_v, rows_v, gsems, osems):
    wid = lax.axis_index("s") * NUM_CORES + lax.axis_index("c")
    base = wid * BPW
    pltpu.sync_copy(idx_hbm.at[pl.ds(base, BPW)], idx_v)

    def start_gather(c):
        return pltpu.async_copy(
            table_hbm.at[idx_v.at[pl.ds(c * CHUNK, CHUNK)]],
            rows_v.at[c % NBUF],
            gsems[c % NBUF],
        )

    gathers = [start_gather(c) for c in range(NBUF)]
    outs = [None] * NCHUNK
    for c in range(NCHUNK):
        b = c % NBUF
        gathers[b].wait()
        outs[c] = pltpu.async_copy(
            rows_v.at[b], out_hbm.at[pl.ds(base + c * CHUNK, CHUNK)], osems[b]
        )
        if c + NBUF < NCHUNK:
            outs[c].wait()
            gathers[b] = start_gather(c + NBUF)
    for c in range(max(NCHUNK - NBUF, 0), NCHUNK):
        outs[c].wait()


def kernel(gcn_embs, offset_ids):
    table_t = gcn_embs.T
    tail = gcn_embs[NMAIN * 128 :].reshape(NTAIL_W * LATENT_DIM)
    flat = _sc_transpose(table_t, tail)
    tbl = flat.reshape(VOCAB, LATENT_DIM)
    flat_ids = offset_ids.reshape(TOTAL)
    out = _sc_gather(tbl, flat_ids)
    return out.reshape(BATCH, N_FIELDS, LATENT_DIM)


# speedup vs baseline: 5.1430x; 2.2242x over previous
import functools

import jax
import jax.numpy as jnp
from jax import lax
from jax.experimental import pallas as pl
from jax.experimental.pallas import tpu as pltpu
from jax.experimental.pallas import tpu_sc as plsc

VOCAB = 1000000
LATENT_DIM = 16
BATCH = 16384
N_FIELDS = 26
TOTAL = BATCH * N_FIELDS

NUM_CORES = 2
NUM_SUBCORES = 16
NW = NUM_CORES * NUM_SUBCORES

_mesh = plsc.VectorSubcoreMesh(core_axis_name="c", subcore_axis_name="s")

NMAIN = 7812
NTAIL_W = 64
T_NB = 4
T_GROUPS = 62


@functools.partial(
    pl.kernel,
    mesh=_mesh,
    out_type=jax.ShapeDtypeStruct((VOCAB * LATENT_DIM,), jnp.float32),
    scratch_types=[
        [pltpu.VMEM((8, 128), jnp.float32)] * T_NB,
        [pltpu.VMEM((8, 128), jnp.float32)] * T_NB,
        [pltpu.VMEM((2048,), jnp.float32)] * T_NB,
        [pltpu.SemaphoreType.DMA] * T_NB,
        [pltpu.SemaphoreType.DMA] * T_NB,
        [pltpu.SemaphoreType.DMA] * T_NB,
    ],
    compiler_params=pltpu.CompilerParams(needs_layout_passes=False),
)
def _sc_transpose(tbl_t_hbm, tail_hbm, flat_hbm, in_a, in_b, rows, sem_a, sem_b, sem_w):
    wid = lax.axis_index("s") * NUM_CORES + lax.axis_index("c")
    lane = lax.iota(jnp.int32, 16)
    lane16 = lane * 16

    def start_reads(b, t):
        @pl.when(t < NMAIN)
        def _():
            pltpu.async_copy(
                tbl_t_hbm.at[pl.ds(0, 8), pl.ds(t * 128, 128)], in_a[b], sem_a[b]
            )
            pltpu.async_copy(
                tbl_t_hbm.at[pl.ds(8, 8), pl.ds(t * 128, 128)], in_b[b], sem_b[b]
            )

    def transpose_tile(b, width):
        for k in range(LATENT_DIM):
            src = in_a[b] if k < 8 else in_b[b]
            for l0 in range(0, width, 16):
                vals = src[k % 8, pl.ds(l0, 16)]
                plsc.store_scatter(rows[b], [lane16 + (l0 * 16 + k)], vals)

    for b in range(T_NB):
        start_reads(b, wid + 32 * b)

    def group(g, _):
        for b in range(T_NB):
            i = g * T_NB + b
            t = wid + 32 * i

            @pl.when(t < NMAIN)
            def _(b=b, i=i, t=t):
                pltpu.make_async_copy(
                    tbl_t_hbm.at[pl.ds(0, 8), pl.ds(t * 128, 128)], in_a[b], sem_a[b]
                ).wait()
                pltpu.make_async_copy(
                    tbl_t_hbm.at[pl.ds(8, 8), pl.ds(t * 128, 128)], in_b[b], sem_b[b]
                ).wait()

                @pl.when(i >= T_NB)
                def _():
                    pltpu.make_async_copy(
                        rows[b], flat_hbm.at[pl.ds((t - 128) * 2048, 2048)], sem_w[b]
                    ).wait()

                transpose_tile(b, 128)
                pltpu.async_copy(
                    rows[b], flat_hbm.at[pl.ds(t * 2048, 2048)], sem_w[b]
                )
                start_reads(b, t + 32 * T_NB)

        return _

    lax.fori_loop(0, T_GROUPS, group, None)

    for b in range(T_NB):
        pltpu.make_async_copy(
            rows[b], flat_hbm.at[pl.ds(0, 2048)], sem_w[b]
        ).wait()

    @pl.when(wid == NMAIN % 32)
    def _():
        pltpu.sync_copy(tail_hbm, rows[0].at[pl.ds(0, NTAIL_W * LATENT_DIM)])
        pltpu.sync_copy(
            rows[0].at[pl.ds(0, NTAIL_W * LATENT_DIM)],
            flat_hbm.at[pl.ds(NMAIN * 2048, NTAIL_W * LATENT_DIM)],
        )


G_IDS = 1024
NTASK = N_FIELDS * 16
TPW = NTASK // NW


@functools.partial(
    pl.kernel,
    mesh=_mesh,
    out_type=jax.ShapeDtypeStruct((N_FIELDS, 2, 16, 8192), jnp.float32),
    scratch_types=[
        [pltpu.VMEM((G_IDS,), jnp.int32)] * 2,
        [pltpu.VMEM((G_IDS, LATENT_DIM), jnp.float32)] * 2,
        [pltpu.VMEM((2, 8192), jnp.float32)] * 2,
        [pltpu.SemaphoreType.DMA] * 2,
        [pltpu.SemaphoreType.DMA] * 2,
    ],
    compiler_params=pltpu.CompilerParams(
        use_tc_tiling_on_sc=False, needs_layout_passes=False
    ),
)
def _sc_gather(table_hbm, idx_hbm, out_hbm, idx_v, rows_v, tbuf, gsems, wsems):
    wid = lax.axis_index("s") * NUM_CORES + lax.axis_index("c")
    lane = lax.iota(jnp.int32, 16)

    def fetch(i, b):
        t = wid + 32 * i
        j = t >> 4
        cbg = t & 15
        pltpu.sync_copy(
            idx_hbm.at[pl.ds(j * BATCH + cbg * G_IDS, G_IDS)], idx_v[b]
        )
        pltpu.async_copy(table_hbm.at[idx_v[b]], rows_v[b], gsems[b])

    def transpose_task(b):
        def cbl_body(cbl, carry):
            for k in range(LATENT_DIM):
                tr, r = divmod(k, 8)
                col = jnp.full((16,), k, jnp.int32)
                for lb in range(8):
                    row_idx = lane + (cbl * 128 + lb * 16)
                    vals = plsc.load_gather(rows_v[b], [row_idx, col])
                    tbuf[b][tr, pl.ds(cbl * 1024 + r * 128 + lb * 16, 16)] = vals
            return carry

        lax.fori_loop(0, 8, cbl_body, None)

    def write_task(i, b):
        t = wid + 32 * i
        j = t >> 4
        cbg = t & 15
        for tr in range(2):
            pltpu.async_copy(tbuf[b].at[tr], out_hbm.at[j, tr, cbg], wsems[b])

    def wait_write(b):
        for _ in range(2):
            pltpu.make_async_copy(tbuf[b].at[0], out_hbm.at[0, 0, 0], wsems[b]).wait()

    fetch(0, 0)

    def group(g, _):
        for b in range(2):
            i = g * 2 + b
            fetch(i + 1, 1 - b)
            pltpu.make_async_copy(
                table_hbm.at[idx_v[b]], rows_v[b], gsems[b]
            ).wait()

            @pl.when(i >= 2)
            def _(b=b):
                wait_write(b)

            transpose_task(b)
            write_task(i, b)
        return _

    lax.fori_loop(0, (TPW - 1) // 2, group, None)

    pltpu.make_async_copy(table_hbm.at[idx_v[0]], rows_v[0], gsems[0]).wait()
    wait_write(0)
    transpose_task(0)
    write_task(TPW - 1, 0)
    wait_write(1)
    wait_write(0)


def kernel(gcn_embs, offset_ids):
    table_t = gcn_embs.T
    tail = gcn_embs[NMAIN * 128 :].reshape(NTAIL_W * LATENT_DIM)
    flat = _sc_transpose(table_t, tail)
    tbl = flat.reshape(VOCAB, LATENT_DIM)
    ids_j = offset_ids.T.reshape(TOTAL)
    out5 = _sc_gather(tbl, ids_j)
    out = (
        out5.reshape(N_FIELDS, 2, 128, 8, 128)
        .transpose(2, 4, 0, 1, 3)
        .reshape(BATCH, N_FIELDS, LATENT_DIM)
    )
    return out


# speedup vs baseline: 5.2998x; 1.0305x over previous
import functools

import jax
import jax.numpy as jnp
from jax import lax
from jax.experimental import pallas as pl
from jax.experimental.pallas import tpu as pltpu
from jax.experimental.pallas import tpu_sc as plsc

VOCAB = 1000000
LATENT_DIM = 16
BATCH = 16384
N_FIELDS = 26
TOTAL = BATCH * N_FIELDS

NUM_CORES = 2
NUM_SUBCORES = 16
NW = NUM_CORES * NUM_SUBCORES

_mesh = plsc.VectorSubcoreMesh(core_axis_name="c", subcore_axis_name="s")

NMAIN = 7812
NTAIL_W = 64
T_NB = 4
T_GROUPS = 62


@functools.partial(
    pl.kernel,
    mesh=_mesh,
    out_type=jax.ShapeDtypeStruct((VOCAB * LATENT_DIM,), jnp.float32),
    scratch_types=[
        [pltpu.VMEM((8, 128), jnp.float32)] * T_NB,
        [pltpu.VMEM((8, 128), jnp.float32)] * T_NB,
        [pltpu.VMEM((2048,), jnp.float32)] * T_NB,
        [pltpu.SemaphoreType.DMA] * T_NB,
        [pltpu.SemaphoreType.DMA] * T_NB,
        [pltpu.SemaphoreType.DMA] * T_NB,
    ],
    compiler_params=pltpu.CompilerParams(needs_layout_passes=False),
)
def _sc_transpose(tbl_t_hbm, tail_hbm, flat_hbm, in_a, in_b, rows, sem_a, sem_b, sem_w):
    wid = lax.axis_index("s") * NUM_CORES + lax.axis_index("c")
    lane = lax.iota(jnp.int32, 16)
    lane16 = lane * 16

    def start_reads(b, t):
        @pl.when(t < NMAIN)
        def _():
            pltpu.async_copy(
                tbl_t_hbm.at[pl.ds(0, 8), pl.ds(t * 128, 128)], in_a[b], sem_a[b]
            )
            pltpu.async_copy(
                tbl_t_hbm.at[pl.ds(8, 8), pl.ds(t * 128, 128)], in_b[b], sem_b[b]
            )

    def transpose_tile(b, width):
        for k in range(LATENT_DIM):
            src = in_a[b] if k < 8 else in_b[b]
            for l0 in range(0, width, 16):
                vals = src[k % 8, pl.ds(l0, 16)]
                plsc.store_scatter(rows[b], [lane16 + (l0 * 16 + k)], vals)

    for b in range(T_NB):
        start_reads(b, wid + 32 * b)

    def group(g, _):
        for b in range(T_NB):
            i = g * T_NB + b
            t = wid + 32 * i

            @pl.when(t < NMAIN)
            def _(b=b, i=i, t=t):
                pltpu.make_async_copy(
                    tbl_t_hbm.at[pl.ds(0, 8), pl.ds(t * 128, 128)], in_a[b], sem_a[b]
                ).wait()
                pltpu.make_async_copy(
                    tbl_t_hbm.at[pl.ds(8, 8), pl.ds(t * 128, 128)], in_b[b], sem_b[b]
                ).wait()

                @pl.when(i >= T_NB)
                def _():
                    pltpu.make_async_copy(
                        rows[b], flat_hbm.at[pl.ds((t - 128) * 2048, 2048)], sem_w[b]
                    ).wait()

                transpose_tile(b, 128)
                pltpu.async_copy(
                    rows[b], flat_hbm.at[pl.ds(t * 2048, 2048)], sem_w[b]
                )
                start_reads(b, t + 32 * T_NB)

        return _

    lax.fori_loop(0, T_GROUPS, group, None)

    for b in range(T_NB):
        pltpu.make_async_copy(
            rows[b], flat_hbm.at[pl.ds(0, 2048)], sem_w[b]
        ).wait()

    @pl.when(wid == NMAIN % 32)
    def _():
        pltpu.sync_copy(tail_hbm, rows[0].at[pl.ds(0, NTAIL_W * LATENT_DIM)])
        pltpu.sync_copy(
            rows[0].at[pl.ds(0, NTAIL_W * LATENT_DIM)],
            flat_hbm.at[pl.ds(NMAIN * 2048, NTAIL_W * LATENT_DIM)],
        )


G_IDS = 1024
NTASK = N_FIELDS * 16
TPW = NTASK // NW


@functools.partial(
    pl.kernel,
    mesh=_mesh,
    out_type=jax.ShapeDtypeStruct((N_FIELDS, 2, 16, 8192), jnp.float32),
    scratch_types=[
        pltpu.VMEM((TPW * G_IDS,), jnp.int32),
        [pltpu.VMEM((G_IDS, LATENT_DIM), jnp.float32)] * 2,
        [pltpu.VMEM((2, 8192), jnp.float32)] * 2,
        [pltpu.SemaphoreType.DMA] * 2,
        [pltpu.SemaphoreType.DMA] * 2,
        pltpu.SemaphoreType.DMA,
    ],
    compiler_params=pltpu.CompilerParams(
        use_tc_tiling_on_sc=False, needs_layout_passes=False
    ),
)
def _sc_gather(table_hbm, idx_hbm, out_hbm, idx_v, rows_v, tbuf, gsems, wsems, isem):
    wid = lax.axis_index("s") * NUM_CORES + lax.axis_index("c")
    lane = lax.iota(jnp.int32, 16)

    pltpu.async_copy(
        idx_hbm.at[pl.ds(wid * TPW * G_IDS, TPW * G_IDS)], idx_v, isem
    )
    pltpu.make_async_copy(
        idx_hbm.at[pl.ds(wid * TPW * G_IDS, TPW * G_IDS)], idx_v, isem
    ).wait()

    def fetch(i, b):
        pltpu.async_copy(
            table_hbm.at[idx_v.at[pl.ds(i * G_IDS, G_IDS)]], rows_v[b], gsems[b]
        )

    def transpose_task(b):
        def cbl_body(cbl, carry):
            w_lb = [lane + (cbl * 128 + lb * 16) for lb in range(8)]
            for k in range(LATENT_DIM):
                tr, r = divmod(k, 8)
                col = jnp.full((16,), k, jnp.int32)
                for lb in range(8):
                    vals = plsc.load_gather(rows_v[b], [w_lb[lb], col])
                    tbuf[b][tr, pl.ds(cbl * 1024 + r * 128 + lb * 16, 16)] = vals
            return carry

        lax.fori_loop(0, 8, cbl_body, None)

    def write_task(i, b):
        t = wid * TPW + i
        j = t >> 4
        cbg = t & 15
        for tr in range(2):
            pltpu.async_copy(tbuf[b].at[tr], out_hbm.at[j, tr, cbg], wsems[b])

    def wait_write(b):
        for _ in range(2):
            pltpu.make_async_copy(tbuf[b].at[0], out_hbm.at[0, 0, 0], wsems[b]).wait()

    fetch(0, 0)

    def group(g, _):
        for b in range(2):
            i = g * 2 + b
            fetch(i + 1, 1 - b)
            pltpu.make_async_copy(
                table_hbm.at[idx_v.at[pl.ds(0, G_IDS)]], rows_v[b], gsems[b]
            ).wait()

            @pl.when(i >= 2)
            def _(b=b):
                wait_write(b)

            transpose_task(b)
            write_task(i, b)
        return _

    lax.fori_loop(0, (TPW - 1) // 2, group, None)

    pltpu.make_async_copy(
        table_hbm.at[idx_v.at[pl.ds(0, G_IDS)]], rows_v[0], gsems[0]
    ).wait()
    wait_write(0)
    transpose_task(0)
    write_task(TPW - 1, 0)
    wait_write(1)
    wait_write(0)


def kernel(gcn_embs, offset_ids):
    table_t = gcn_embs.T
    tail = gcn_embs[NMAIN * 128 :].reshape(NTAIL_W * LATENT_DIM)
    flat = _sc_transpose(table_t, tail)
    tbl = flat.reshape(VOCAB, LATENT_DIM)
    ids_j = offset_ids.T.reshape(TOTAL)
    out5 = _sc_gather(tbl, ids_j)
    out = (
        out5.reshape(N_FIELDS, 2, 128, 8, 128)
        .transpose(2, 4, 0, 1, 3)
        .reshape(BATCH, N_FIELDS, LATENT_DIM)
    )
    return out
